# baseline (speedup 1.0000x reference)
import functools

import jax
import jax.numpy as jnp
from jax import lax
from jax.experimental import pallas as pl
from jax.experimental.pallas import tpu as pltpu
from jax.experimental.pallas import tpu_sc as plsc


def kernel(x, emb_table, pos_embd):
    B, S = x.shape
    V, D = emb_table.shape
    info = plsc.get_sparse_core_info()
    NC, NS, L = info.num_cores, info.num_subcores, info.num_lanes
    NW = NC * NS
    elems_per_w = B // NW

    mesh = plsc.VectorSubcoreMesh(core_axis_name="c", subcore_axis_name="s")

    @functools.partial(
        pl.kernel,
        mesh=mesh,
        compiler_params=pltpu.CompilerParams(use_tc_tiling_on_sc=False),
        out_type=jax.ShapeDtypeStruct((B, S, D), jnp.float32),
        scratch_types=[
            pltpu.VMEM((S,), jnp.int32),
            pltpu.VMEM((S, D), jnp.float32),
            pltpu.VMEM((S, D), jnp.float32),
            pltpu.SemaphoreType.DMA,
        ],
    )
    def emb_kernel(x_hbm, table_hbm, pos_hbm, out_hbm, idx_v, rows_v, pos_v, sem):
        wid = lax.axis_index("s") * NC + lax.axis_index("c")
        pltpu.sync_copy(pos_hbm, pos_v)

        def elem_body(e, carry):
            b = wid * elems_per_w + e
            pltpu.sync_copy(x_hbm.at[b], idx_v)
            pltpu.async_copy(table_hbm.at[idx_v], rows_v, sem).wait()

            def row_body(r, carry2):
                for d in range(D // L):
                    sl = pl.ds(d * L, L)
                    rows_v[r, sl] = rows_v[r, sl] + pos_v[r, sl]
                return carry2

            lax.fori_loop(0, S, row_body, 0)
            pltpu.sync_copy(rows_v, out_hbm.at[b])
            return carry

        lax.fori_loop(0, elems_per_w, elem_body, 0)

    return emb_kernel(x.astype(jnp.int32), emb_table, pos_embd)


# speedup vs baseline: 1.1919x; 1.1919x over previous
import functools

import jax
import jax.numpy as jnp
from jax import lax
from jax.experimental import pallas as pl
from jax.experimental.pallas import tpu as pltpu
from jax.experimental.pallas import tpu_sc as plsc

_NBUF = 4


def kernel(x, emb_table, pos_embd):
    B, S = x.shape
    V, D = emb_table.shape
    info = plsc.get_sparse_core_info()
    NC, NS, L = info.num_cores, info.num_subcores, info.num_lanes
    NW = NC * NS
    EPW = B // NW

    mesh = plsc.VectorSubcoreMesh(core_axis_name="c", subcore_axis_name="s")

    @functools.partial(
        pl.kernel,
        mesh=mesh,
        compiler_params=pltpu.CompilerParams(use_tc_tiling_on_sc=False),
        out_type=jax.ShapeDtypeStruct((B, S, D), jnp.float32),
        scratch_types=[
            pltpu.VMEM((EPW, S), jnp.int32),
            pltpu.VMEM((_NBUF, S, D), jnp.float32),
            pltpu.VMEM((S, D), jnp.float32),
        ]
        + [pltpu.SemaphoreType.DMA] * (2 * _NBUF),
    )
    def emb_kernel(x_hbm, table_hbm, pos_hbm, out_hbm, idx_all, rows_v, pos_v, *sems):
        gsem = sems[:_NBUF]
        ssem = sems[_NBUF:]
        wid = lax.axis_index("s") * NC + lax.axis_index("c")
        e0 = wid * EPW

        pltpu.sync_copy(pos_hbm, pos_v)
        pltpu.sync_copy(x_hbm.at[pl.ds(e0, EPW)], idx_all)

        def start_gather(c, b):
            pltpu.async_copy(table_hbm.at[idx_all.at[c]], rows_v.at[b], gsem[b])

        def wait_gather(c, b):
            pltpu.make_async_copy(
                table_hbm.at[idx_all.at[c]], rows_v.at[b], gsem[b]
            ).wait()

        def start_store(c, b):
            pltpu.async_copy(rows_v.at[b], out_hbm.at[e0 + c], ssem[b])

        def wait_store(c, b):
            pltpu.make_async_copy(rows_v.at[b], out_hbm.at[e0 + c], ssem[b]).wait()

        start_gather(0, 0)
        start_gather(1, 1)

        @pl.loop(0, EPW, step=_NBUF)
        def ring(g):
            for k in range(_NBUF):
                c = g + k
                b = k

                wait_gather(c, b)

                @pl.loop(0, S)
                def row_add(r):
                    for d in range(D // L):
                        sl = pl.ds(d * L, L)
                        plsc.addupdate(rows_v.at[b, r, sl], pos_v[r, sl])

                start_store(c, b)

                nb = (k + 2) % _NBUF

                @pl.when(c >= 2)
                def _():
                    wait_store(c + 2 - _NBUF, nb)

                @pl.when(c + 2 < EPW)
                def _():
                    start_gather(c + 2, nb)

        wait_store(EPW - 2, (EPW - 2) % _NBUF)
        wait_store(EPW - 1, (EPW - 1) % _NBUF)

    return emb_kernel(x.astype(jnp.int32), emb_table, pos_embd)
